# Initial kernel scaffold; baseline (speedup 1.0000x reference)
import jax
import jax.numpy as jnp
from jax.experimental import pallas as pl


def kernel(x, edge_index, node_time, edge_time, node_out_degree, Wt, bt, W1, b1, Wd, bd, Wc, bc, Wq, bq, Wk, bk, Wv, bv, We, be, Ws, bs, Wo, bo):
    raise NotImplementedError("write your pallas kernel here")


# speedup vs baseline: 15.9325x; 15.9325x over previous
import functools
import jax
import jax.numpy as jnp
from jax import lax
from jax.experimental import pallas as pl
from jax.experimental.pallas import tpu as pltpu
from jax.experimental.pallas import tpu_sc as plsc

N = 10000
E = 320000
D_IN = 128
D_H = 32
HEADS = 2
C = 16
HC = HEADS * C

NW = 32
EPW = E // NW
GB = 400
NB_G = EPW // GB
MB = 2000
NB_M = EPW // MB
SB = 400
NB_S = EPW // SB
NPAD = 10240
NSL = NPAD // 16
ROWS_PER_SC = N // 16
AW = 48
NEG = -3.0e38

_mesh = plsc.VectorSubcoreMesh(core_axis_name="c", subcore_axis_name="s")
_SC_PARAMS = pltpu.CompilerParams(needs_layout_passes=False, use_tc_tiling_on_sc=False)


def _k1_body(x_ref, deg_ref, W1, b1, Wd, bd, Wc, bc, Wq, bq, Wk, bk, Wv, bv,
             Ws, bs, kv_ref, q_ref, skip_ref):
    h1 = jax.nn.relu(x_ref[...] @ W1[...] + b1[...])
    dg = jax.nn.relu(deg_ref[...] * Wd[...][0] + bd[...])
    hc = h1 @ Wc[...][:D_H, :] + dg @ Wc[...][D_H:, :] + bc[...]
    q_ref[...] = hc @ Wq[...] + bq[...]
    kv_ref[:, :HC] = hc @ Wk[...] + bk[...]
    kv_ref[:, HC:] = hc @ Wv[...] + bv[...]
    skip_ref[...] = hc @ Ws[...] + bs[...]


def _k1(x, deg, W1, b1, Wd, bd, Wc, bc, Wq, bq, Wk, bk, Wv, bv, Ws, bs):
    R = 1000
    full = lambda shp: pl.BlockSpec(shp, lambda i: (0,) * len(shp))
    return pl.pallas_call(
        _k1_body,
        grid=(N // R,),
        in_specs=[
            pl.BlockSpec((R, D_IN), lambda i: (i, 0)),
            pl.BlockSpec((R, 1), lambda i: (i, 0)),
            full((D_IN, D_H)), full((D_H,)),
            full((1, 8)), full((8,)),
            full((D_H + 8, D_H)), full((D_H,)),
            full((D_H, HC)), full((HC,)),
            full((D_H, HC)), full((HC,)),
            full((D_H, HC)), full((HC,)),
            full((D_H, HC)), full((HC,)),
        ],
        out_specs=[
            pl.BlockSpec((R, 2 * HC), lambda i: (i, 0)),
            pl.BlockSpec((R, HC), lambda i: (i, 0)),
            pl.BlockSpec((R, HC), lambda i: (i, 0)),
        ],
        out_shape=[
            jax.ShapeDtypeStruct((N, 2 * HC), jnp.float32),
            jax.ShapeDtypeStruct((N, HC), jnp.float32),
            jax.ShapeDtypeStruct((N, HC), jnp.float32),
        ],
    )(x, deg, W1, b1, Wd, bd, Wc, bc, Wq, bq, Wk, bk, Wv, bv, Ws, bs)


@functools.partial(
    pl.kernel,
    out_type=[
        jax.ShapeDtypeStruct((E, 2 * HC), jnp.float32),
        jax.ShapeDtypeStruct((E, HC), jnp.float32),
        jax.ShapeDtypeStruct((E,), jnp.float32),
    ],
    mesh=_mesh,
    compiler_params=_SC_PARAMS,
    scratch_types=[
        pltpu.VMEM((GB,), jnp.int32),
        pltpu.VMEM((GB,), jnp.int32),
        pltpu.VMEM((GB, 2 * HC), jnp.float32),
        pltpu.VMEM((GB, HC), jnp.float32),
        pltpu.VMEM((GB,), jnp.float32),
        pltpu.VMEM((N,), jnp.float32),
        pltpu.SemaphoreType.DMA,
        pltpu.SemaphoreType.DMA,
    ],
)
def _k2(kv_hbm, q_hbm, nt_hbm, src_hbm, dst_hbm,
        kvs_hbm, qd_hbm, ts_hbm,
        src_v, dst_v, kv_v, q_v, t_v, nt_v, sem1, sem2):
    wid = lax.axis_index("s") * 2 + lax.axis_index("c")
    base0 = wid * EPW
    pltpu.sync_copy(nt_hbm, nt_v)

    def batch(i, _):
        base = base0 + i * GB
        pltpu.sync_copy(src_hbm.at[pl.ds(base, GB)], src_v)
        pltpu.sync_copy(dst_hbm.at[pl.ds(base, GB)], dst_v)
        cp1 = pltpu.async_copy(kv_hbm.at[src_v], kv_v, sem1)
        cp2 = pltpu.async_copy(q_hbm.at[dst_v], q_v, sem2)
        cp1.wait()
        cp2.wait()
        for j in range(GB // 16):
            idx = src_v[pl.ds(j * 16, 16)]
            t_v[pl.ds(j * 16, 16)] = plsc.load_gather(nt_v, [idx])
        pltpu.sync_copy(kv_v, kvs_hbm.at[pl.ds(base, GB), :])
        pltpu.sync_copy(q_v, qd_hbm.at[pl.ds(base, GB), :])
        pltpu.sync_copy(t_v, ts_hbm.at[pl.ds(base, GB)])
        return 0

    lax.fori_loop(0, NB_G, batch, 0)


def _k3_body(kvs_ref, qd_ref, ts_ref, et_ref, Wt, bt, We, be,
             vj_ref, al_ref):
    rel = ts_ref[...] - et_ref[...]
    enc = jnp.cos(rel * Wt[...] + bt[...])
    e = enc @ We[...] + be[...]
    k = kvs_ref[:, :HC] + e
    vj_ref[...] = kvs_ref[:, HC:] + e
    qk = qd_ref[...] * k
    a0 = jnp.sum(qk[:, :C], axis=1) * 0.25
    a1 = jnp.sum(qk[:, C:], axis=1) * 0.25
    al_ref[...] = jnp.stack([a0, a1], axis=1)


def _k3(kvs, qd, ts, et, Wt, bt, We, be):
    R = 4000
    full = lambda shp: pl.BlockSpec(shp, lambda i: (0,) * len(shp))
    return pl.pallas_call(
        _k3_body,
        grid=(E // R,),
        in_specs=[
            pl.BlockSpec((R, 2 * HC), lambda i: (i, 0)),
            pl.BlockSpec((R, HC), lambda i: (i, 0)),
            pl.BlockSpec((R, 1), lambda i: (i, 0)),
            pl.BlockSpec((R, 1), lambda i: (i, 0)),
            full((1, HC)), full((HC,)),
            full((HC, HC)), full((HC,)),
        ],
        out_specs=[
            pl.BlockSpec((R, HC), lambda i: (i, 0)),
            pl.BlockSpec((R, HEADS), lambda i: (i, 0)),
        ],
        out_shape=[
            jax.ShapeDtypeStruct((E, HC), jnp.float32),
            jax.ShapeDtypeStruct((E, HEADS), jnp.float32),
        ],
    )(kvs, qd, ts, et, Wt, bt, We, be)


@functools.partial(
    pl.kernel,
    out_type=jax.ShapeDtypeStruct((NW, 2, NPAD), jnp.float32),
    mesh=_mesh,
    compiler_params=_SC_PARAMS,
    scratch_types=[
        pltpu.VMEM((MB, HEADS), jnp.float32),
        pltpu.VMEM((MB,), jnp.int32),
        pltpu.VMEM((NPAD,), jnp.float32),
        pltpu.VMEM((NPAD,), jnp.float32),
    ],
)
def _k4a(al_hbm, dst_hbm, amax_hbm, al_v, dst_v, t0_v, t1_v):
    wid = lax.axis_index("s") * 2 + lax.axis_index("c")
    base0 = wid * EPW

    def init(i, _):
        t0_v[pl.ds(i * 16, 16)] = jnp.full((16,), NEG, jnp.float32)
        t1_v[pl.ds(i * 16, 16)] = jnp.full((16,), NEG, jnp.float32)
        return 0

    lax.fori_loop(0, NPAD // 16, init, 0)

    IOT = lax.broadcasted_iota(jnp.int32, (16,), 0)
    ZC = jnp.zeros((16,), jnp.int32)
    OC = jnp.ones((16,), jnp.int32)

    def batch(i, _):
        base = base0 + i * MB
        pltpu.sync_copy(al_hbm.at[pl.ds(base, MB), :], al_v)
        pltpu.sync_copy(dst_hbm.at[pl.ds(base, MB)], dst_v)

        def grp(j, _):
            didx = dst_v[pl.ds(j * 16, 16)]
            rows = IOT + j * 16
            a0 = plsc.load_gather(al_v, [rows, ZC])
            a1 = plsc.load_gather(al_v, [rows, OC])

            def upd(tab, a):
                for l in range(16):
                    cur = plsc.load_gather(tab, [didx])
                    plsc.store_scatter(tab, [didx], jnp.maximum(cur, a),
                                       mask=IOT == l)

            upd(t0_v, a0)
            upd(t1_v, a1)
            return 0

        lax.fori_loop(0, MB // 16, grp, 0)
        return 0

    lax.fori_loop(0, NB_M, batch, 0)
    pltpu.sync_copy(t0_v, amax_hbm.at[wid, 0])
    pltpu.sync_copy(t1_v, amax_hbm.at[wid, 1])


@functools.partial(
    pl.kernel,
    out_type=jax.ShapeDtypeStruct((2, N, AW), jnp.float32),
    mesh=_mesh,
    compiler_params=_SC_PARAMS,
    scratch_types=[
        pltpu.VMEM((NPAD,), jnp.float32),
        pltpu.VMEM((NPAD,), jnp.float32),
        pltpu.VMEM((NSL,), jnp.float32),
        pltpu.VMEM((NSL,), jnp.float32),
        pltpu.VMEM((SB, HC), jnp.float32),
        pltpu.VMEM((SB, HEADS), jnp.float32),
        pltpu.VMEM((SB,), jnp.int32),
        pltpu.VMEM((SB,), jnp.float32),
        pltpu.VMEM((SB,), jnp.float32),
        pltpu.VMEM((SB, AW), jnp.float32),
        pltpu.VMEM_SHARED((N, AW), jnp.float32),
        pltpu.VMEM_SHARED((2, NPAD), jnp.float32),
    ],
)
def _k4b(vj_hbm, al_hbm, dst_hbm, amax_hbm, out_hbm,
         am0_v, am1_v, acc_v, tmp_v, vj_v, al_v, dst_v, w0_v, w1_v, msg_v,
         agg_sh, amax_sh):
    cid = lax.axis_index("c")
    sid = lax.axis_index("s")
    wid = sid * 2 + cid

    for h in range(2):
        pltpu.sync_copy(amax_hbm.at[0, h, pl.ds(sid * NSL, NSL)], acc_v)

        def red(t, _):
            pltpu.sync_copy(amax_hbm.at[t, h, pl.ds(sid * NSL, NSL)], tmp_v)

            def vmax(j, _):
                sl = pl.ds(j * 16, 16)
                acc_v[sl] = jnp.maximum(acc_v[sl], tmp_v[sl])
                return 0

            lax.fori_loop(0, NSL // 16, vmax, 0)
            return 0

        lax.fori_loop(1, NW, red, 0)
        pltpu.sync_copy(acc_v, amax_sh.at[h, pl.ds(sid * NSL, NSL)])

    def zrow(e, _):
        msg_v[e, pl.ds(0, 16)] = jnp.zeros((16,), jnp.float32)
        msg_v[e, pl.ds(16, 16)] = jnp.zeros((16,), jnp.float32)
        msg_v[e, pl.ds(32, 16)] = jnp.zeros((16,), jnp.float32)
        return 0

    lax.fori_loop(0, SB, zrow, 0)
    r0 = sid * ROWS_PER_SC
    pltpu.sync_copy(msg_v, agg_sh.at[pl.ds(r0, SB), :])
    pltpu.sync_copy(msg_v.at[pl.ds(0, ROWS_PER_SC - SB), :],
                    agg_sh.at[pl.ds(r0 + SB, ROWS_PER_SC - SB), :])

    plsc.subcore_barrier()
    pltpu.sync_copy(amax_sh.at[0], am0_v)
    pltpu.sync_copy(amax_sh.at[1], am1_v)

    base0 = wid * EPW
    ZC = jnp.zeros((16,), jnp.int32)
    OC = jnp.ones((16,), jnp.int32)
    C32 = jnp.full((16,), 32, jnp.int32)
    C33 = jnp.full((16,), 33, jnp.int32)
    IOT = lax.broadcasted_iota(jnp.int32, (16,), 0)

    def batch(i, _):
        base = base0 + i * SB
        pltpu.sync_copy(vj_hbm.at[pl.ds(base, SB), :], vj_v)
        pltpu.sync_copy(al_hbm.at[pl.ds(base, SB), :], al_v)
        pltpu.sync_copy(dst_hbm.at[pl.ds(base, SB)], dst_v)
        for j in range(SB // 16):
            sl = pl.ds(j * 16, 16)
            didx = dst_v[sl]
            rows = IOT + (j * 16)
            m0 = plsc.load_gather(am0_v, [didx])
            m1 = plsc.load_gather(am1_v, [didx])
            a0 = plsc.load_gather(al_v, [rows, ZC])
            a1 = plsc.load_gather(al_v, [rows, OC])
            w0g = jnp.exp(a0 - m0)
            w1g = jnp.exp(a1 - m1)
            plsc.store_scatter(msg_v, [rows, C32], w0g)
            plsc.store_scatter(msg_v, [rows, C33], w1g)
            for l in range(16):
                e = j * 16 + l
                msg_v[e, pl.ds(0, 16)] = vj_v[e, pl.ds(0, 16)] * w0g[l]
                msg_v[e, pl.ds(16, 16)] = vj_v[e, pl.ds(16, 16)] * w1g[l]

        pltpu.sync_copy(msg_v, agg_sh.at[dst_v], add=True)
        return 0

    lax.fori_loop(0, NB_S, batch, 0)
    plsc.subcore_barrier()
    pltpu.sync_copy(agg_sh.at[pl.ds(r0, ROWS_PER_SC), :],
                    out_hbm.at[cid, pl.ds(r0, ROWS_PER_SC), :])


def _k5_body(a_ref, b_ref, skip_ref, Wo, bo, out_ref):
    a = a_ref[...] + b_ref[...]
    s0 = a[:, 32:33] + 1e-16
    s1 = a[:, 33:34] + 1e-16
    h0 = a[:, 0:16] / s0
    h1 = a[:, 16:32] / s1
    h = jnp.concatenate([h0, h1], axis=1) + skip_ref[...]
    o = h @ Wo[...] + bo[...]
    m = jnp.max(o, axis=1, keepdims=True)
    lse = m + jnp.log(jnp.sum(jnp.exp(o - m), axis=1, keepdims=True))
    out_ref[...] = o - lse


def _k5(agg_a, agg_b, skip, Wo, bo):
    R = 1000
    full = lambda shp: pl.BlockSpec(shp, lambda i: (0,) * len(shp))
    return pl.pallas_call(
        _k5_body,
        grid=(N // R,),
        in_specs=[
            pl.BlockSpec((R, AW), lambda i: (i, 0)),
            pl.BlockSpec((R, AW), lambda i: (i, 0)),
            pl.BlockSpec((R, HC), lambda i: (i, 0)),
            full((HC, 2)), full((2,)),
        ],
        out_specs=pl.BlockSpec((R, 2), lambda i: (i, 0)),
        out_shape=jax.ShapeDtypeStruct((N, 2), jnp.float32),
    )(agg_a, agg_b, skip, Wo, bo)


def kernel(x, edge_index, node_time, edge_time, node_out_degree,
           Wt, bt, W1, b1, Wd, bd, Wc, bc, Wq, bq, Wk, bk, Wv, bv,
           We, be, Ws, bs, Wo, bo):
    src = edge_index[0]
    dst = edge_index[1]
    kv, q, skip = _k1(x, node_out_degree, W1, b1, Wd, bd, Wc, bc,
                      Wq, bq, Wk, bk, Wv, bv, Ws, bs)
    kvs, qd, ts = _k2(kv, q, node_time, src, dst)
    vj, al = _k3(kvs, qd, ts.reshape(E, 1), edge_time, Wt, bt, We, be)
    amax = _k4a(al, dst)
    agg = _k4b(vj, al, dst, amax)
    return _k5(agg[0], agg[1], skip, Wo, bo)
